# Initial kernel scaffold; baseline (speedup 1.0000x reference)
import jax
import jax.numpy as jnp
from jax.experimental import pallas as pl


def kernel(hidden_states, weight):
    raise NotImplementedError("write your pallas kernel here")


# speedup vs baseline: 7.4399x; 7.4399x over previous
import functools

import jax
import jax.numpy as jnp
from jax.experimental import pallas as pl
from jax.experimental.pallas import tpu as pltpu

HIDDEN = 768
EXPERTS = 64
TOPK = 8
BLOCK = 2048
NEG_INF = float("-inf")


def _router_block_kernel(hs_ref, w_ref, scores_ref, idx_ref):
    logits = jax.lax.dot_general(
        hs_ref[...], w_ref[...],
        dimension_numbers=(((1,), (1,)), ((), ())),
        preferred_element_type=jnp.float32,
    )
    iota = jax.lax.broadcasted_iota(jnp.int32, logits.shape, 1)

    x = logits
    sel = jnp.zeros(logits.shape, dtype=jnp.bool_)
    idx_cols = []
    m0 = None
    for _ in range(TOPK):
        m = jnp.max(x, axis=1, keepdims=True)
        if m0 is None:
            m0 = m
        idx = jnp.min(jnp.where(x == m, iota, EXPERTS), axis=1, keepdims=True)
        one_hot = iota == idx
        sel = jnp.logical_or(sel, one_hot)
        idx_cols.append(idx)
        x = jnp.where(one_hot, NEG_INF, x)

    e = jnp.where(sel, jnp.exp(logits - m0), 0.0)
    z = jnp.sum(e, axis=1, keepdims=True)
    scores_ref[...] = e / z
    idx_ref[...] = jnp.concatenate(idx_cols, axis=1)


@jax.jit
def kernel(hidden_states, weight):
    hs = hidden_states.reshape(-1, HIDDEN)
    n_tok = hs.shape[0]
    grid = (n_tok // BLOCK,)
    scores, indices = pl.pallas_call(
        _router_block_kernel,
        grid=grid,
        in_specs=[
            pl.BlockSpec((BLOCK, HIDDEN), lambda i: (i, 0)),
            pl.BlockSpec((EXPERTS, HIDDEN), lambda i: (0, 0)),
        ],
        out_specs=[
            pl.BlockSpec((BLOCK, EXPERTS), lambda i: (i, 0)),
            pl.BlockSpec((BLOCK, TOPK), lambda i: (i, 0)),
        ],
        out_shape=[
            jax.ShapeDtypeStruct((n_tok, EXPERTS), jnp.float32),
            jax.ShapeDtypeStruct((n_tok, TOPK), jnp.int32),
        ],
    )(hs, weight)
    return scores, indices
